# baseline (speedup 1.0000x reference)
import dataclasses
import functools
import math

import numpy as np

import jax
import jax.numpy as jnp
from jax import lax
from jax.experimental import pallas as pl
from jax.experimental.pallas import tpu as pltpu
from jax.experimental.pallas import tpu_sc as plsc

_N = 10000
_NPAD = 10240
_E = 320000
_NCORES = 2
_NSUB = 16
_NW = _NCORES * _NSUB
_CH = 64
_NCH = 160
_EPT = _NCH * _CH
_EPAD = _EPT * _NW
_R = 1000
_PREC = lax.Precision.DEFAULT
_DN = (((1,), (1,)), ((), ()))


def _matmul(a, w):
    return lax.dot_general(a, w, _DN, precision=_PREC)

_sc_mesh = plsc.VectorSubcoreMesh(core_axis_name="c", subcore_axis_name="s")

_sc_params = pltpu.CompilerParams()
if "needs_layout_passes" in pltpu.CompilerParams.__dataclass_fields__:
    _sc_params = dataclasses.replace(_sc_params, needs_layout_passes=False)


@functools.partial(
    pl.kernel,
    out_type=jax.ShapeDtypeStruct((_NW, _NPAD), jnp.float32),
    mesh=_sc_mesh,
    compiler_params=_sc_params,
    scratch_types=[
        pltpu.VMEM((_NPAD,), jnp.float32),
        pltpu.VMEM((_NCH, _CH), jnp.int32),
    ],
)
def _sc_degree_hist(col_hbm, out_hbm, hist_v, cslab):
    c = lax.axis_index("c")
    s = lax.axis_index("s")
    wid = c * _NSUB + s

    @pl.loop(0, _NPAD // 16)
    def _(i):
        hist_v[pl.ds(i * 16, 16)] = jnp.zeros((16,), jnp.float32)

    ones = jnp.ones((16,), jnp.float32)
    pltpu.sync_copy(col_hbm.at[wid], cslab)

    @pl.loop(0, _NCH)
    def _(j):
        @pl.loop(0, _CH // 16)
        def _(k):
            idx = cslab[j, pl.ds(k * 16, 16)]
            plsc.addupdate_scatter(hist_v, [idx], ones)

    pltpu.sync_copy(hist_v, out_hbm.at[wid])


@functools.partial(
    pl.kernel,
    out_type=jax.ShapeDtypeStruct((_NCORES, _NPAD, 128), jnp.float32),
    mesh=_sc_mesh,
    scratch_types=[
        [pltpu.VMEM((_CH, 128), jnp.float32)] * 4,
        [pltpu.VMEM((_CH,), jnp.int32)] * 4,
        [pltpu.VMEM((_CH,), jnp.int32)] * 4,
        [pltpu.SemaphoreType.DMA] * 4,
        [pltpu.SemaphoreType.DMA] * 4,
        [pltpu.SemaphoreType.DMA] * 4,
        [pltpu.SemaphoreType.DMA] * 4,
        pltpu.VMEM_SHARED((_NPAD, 128), jnp.float32),
    ],
)
def _sc_edge_pass(y_hbm, row_hbm, col_hbm, zeros_hbm, out_hbm,
                  gbufs, cbufs, rbufs, gsems, ssems, rsems, csems, acc):
    c = lax.axis_index("c")
    s = lax.axis_index("s")
    wid = c * _NSUB + s

    pltpu.sync_copy(zeros_hbm, gbufs[0])
    zrows = _NPAD // _NSUB

    @pl.loop(0, zrows // _CH)
    def _(i):
        pltpu.sync_copy(gbufs[0], acc.at[pl.ds(s * zrows + i * _CH, _CH)])

    plsc.subcore_barrier()

    def wait_r(k, b):
        pltpu.make_async_copy(
            row_hbm.at[wid, k], rbufs[b], rsems[b]).wait()

    def wait_c(k, b):
        pltpu.make_async_copy(
            col_hbm.at[wid, k], cbufs[b], csems[b]).wait()

    def start_g(b):
        pltpu.async_copy(y_hbm.at[rbufs[b]], gbufs[b], gsems[b])

    def wait_g(b):
        pltpu.make_async_copy(y_hbm.at[rbufs[b]], gbufs[b], gsems[b]).wait()

    def start_s(b):
        pltpu.async_copy(gbufs[b], acc.at[cbufs[b]], ssems[b], add=True)

    def wait_s(b):
        pltpu.make_async_copy(gbufs[b], acc.at[cbufs[b]], ssems[b]).wait()

    for k in range(4):
        pltpu.async_copy(row_hbm.at[wid, k], rbufs[k], rsems[k])
    for k in range(2):
        pltpu.async_copy(col_hbm.at[wid, k], cbufs[k], csems[k])
    for k in range(2):
        wait_r(k, k)
        start_g(k)

    @pl.loop(0, _NCH, step=4)
    def _(j):
        for i in range(4):
            jj = j + i
            wait_g(i)
            wait_c(jj, i)
            start_s(i)

            @pl.when(jj + 4 < _NCH)
            def _():
                pltpu.async_copy(
                    row_hbm.at[wid, jj + 4], rbufs[i], rsems[i])

            @pl.when(jj + 2 < _NCH)
            def _():
                b2 = (i + 2) % 4

                @pl.when(jj >= 2)
                def _():
                    wait_s(b2)

                pltpu.async_copy(
                    col_hbm.at[wid, jj + 2], cbufs[b2], csems[b2])
                wait_r(jj + 2, b2)
                start_g(b2)

    for k in range(4):
        wait_s(k)

    plsc.subcore_barrier()

    orows = _NPAD // _NSUB
    pltpu.sync_copy(acc.at[pl.ds(s * orows, orows)],
                    out_hbm.at[c, pl.ds(s * orows, orows)])


def _ln(x, scale, bias):
    mu = jnp.mean(x, axis=-1, keepdims=True)
    var = jnp.mean((x - mu) ** 2, axis=-1, keepdims=True)
    return (x - mu) * lax.rsqrt(var + 1e-5) * scale + bias


def _full(shape):
    return pl.BlockSpec(shape, lambda i: (0,) * len(shape))


def _rows(shape):
    return pl.BlockSpec(shape, lambda i: (i,) + (0,) * (len(shape) - 1))


def _dinv_body(hist_ref, out_ref):
    total = jnp.sum(hist_ref[...], axis=0) + 2.0
    out_ref[...] = lax.rsqrt(total)[:, None]


def _embed_body(x_ref, w_ref, b_ref, out_ref):
    out_ref[...] = _matmul(x_ref[...], w_ref[...]) + b_ref[...]


def _pre_body(h_ref, dinv_ref, lns_ref, lnb_ref, pinw_ref, pinb_ref,
              sgus_ref, sgub_ref, gcnw_ref, t_ref, y_ref):
    hn = _ln(h_ref[...], lns_ref[...], lnb_ref[...])
    t = _matmul(hn, pinw_ref[...]) + pinb_ref[...]
    t = 0.5 * t * (1.0 + lax.erf(t * (1.0 / math.sqrt(2.0))))
    g = _ln(t, sgus_ref[...], sgub_ref[...])
    y = dinv_ref[...] * _matmul(g, gcnw_ref[...])
    t_ref[...] = t
    y_ref[...] = y


def _post_body(p_ref, y_ref, t_ref, h_ref, dinv_ref, gcnb_ref,
               poutw_ref, poutb_ref, out_ref):
    acc = p_ref[0] + p_ref[1] + 2.0 * y_ref[...]
    g2 = jnp.tanh(dinv_ref[...] * acc + gcnb_ref[...])
    t2 = g2 * t_ref[...]
    out_ref[...] = h_ref[...] + _matmul(t2, poutw_ref[...]) + poutb_ref[...]


def _final_body(h_ref, w_ref, b_ref, out_ref):
    out_ref[...] = _matmul(h_ref[...], w_ref[...]) + b_ref[...]


_G = _N // _R


def _tc(body, out_shapes, in_specs, out_specs):
    return pl.pallas_call(
        body,
        grid=(_G,),
        in_specs=in_specs,
        out_specs=out_specs,
        out_shape=out_shapes,
    )


def kernel(x, edge_index, batch, emb_W, emb_b, ln_s, ln_b, pin_W, pin_b,
           sgu_s, sgu_b, gcn_W, gcn_b, pout_W, pout_b, out_W, out_b):
    f32 = jnp.float32
    row = edge_index[0]
    col = edge_index[1]

    npad = _EPAD - _E
    pidx = np.arange(npad, dtype=np.int32)
    row_p = jnp.concatenate(
        [row, jnp.asarray((pidx * 37) % _N, jnp.int32)]).reshape(_NW, _NCH, _CH)
    col_p = jnp.concatenate(
        [col, jnp.asarray(_N + pidx % (_NPAD - _N), jnp.int32)]
    ).reshape(_NW, _NCH, _CH)
    zeros_blk = jnp.zeros((_CH, 128), f32)

    hist = _sc_degree_hist(col_p)
    dinv = pl.pallas_call(
        _dinv_body,
        out_shape=jax.ShapeDtypeStruct((_NPAD, 1), f32),
    )(hist)

    h = _tc(
        _embed_body,
        jax.ShapeDtypeStruct((_N, 128), f32),
        [_rows((_R, 128)), _full((128, 128)), _full((1, 128))],
        _rows((_R, 128)),
    )(x, emb_W, emb_b[None, :])

    for i in range(2):
        t, y = _tc(
            _pre_body,
            (jax.ShapeDtypeStruct((_N, 128), f32),
             jax.ShapeDtypeStruct((_N, 128), f32)),
            [_rows((_R, 128)), _rows((_R, 1)),
             _full((1, 128)), _full((1, 128)),
             _full((128, 128)), _full((1, 128)),
             _full((1, 128)), _full((1, 128)),
             _full((128, 128))],
            (_rows((_R, 128)), _rows((_R, 128))),
        )(h, dinv, ln_s[i][None, :], ln_b[i][None, :], pin_W[i],
          pin_b[i][None, :], sgu_s[i][None, :], sgu_b[i][None, :],
          gcn_W[i])

        partials = _sc_edge_pass(y, row_p, col_p, zeros_blk)

        h = _tc(
            _post_body,
            jax.ShapeDtypeStruct((_N, 128), f32),
            [pl.BlockSpec((2, _R, 128), lambda i: (0, i, 0)),
             _rows((_R, 128)), _rows((_R, 128)), _rows((_R, 128)),
             _rows((_R, 1)), _full((1, 128)),
             _full((128, 128)), _full((1, 128))],
            _rows((_R, 128)),
        )(partials, y, t, h, dinv, gcn_b[i][None, :], pout_W[i],
          pout_b[i][None, :])

    out = _tc(
        _final_body,
        jax.ShapeDtypeStruct((_N, 64), f32),
        [_rows((_R, 128)), _full((64, 128)), _full((1, 64))],
        _rows((_R, 64)),
    )(h, out_W, out_b[None, :])
    return out


# speedup vs baseline: 1.0581x; 1.0581x over previous
import dataclasses
import functools
import math

import numpy as np

import jax
import jax.numpy as jnp
from jax import lax
from jax.experimental import pallas as pl
from jax.experimental.pallas import tpu as pltpu
from jax.experimental.pallas import tpu_sc as plsc

_N = 10000
_NPAD = 10240
_E = 320000
_NCORES = 2
_NSUB = 16
_NW = _NCORES * _NSUB
_CH = 128
_NRC = _E // _CH
_NB = _NRC // _NW
_NX = _NRC - _NB * _NW
_NCH = _NB + 2
_R = 1000
_PREC = lax.Precision.DEFAULT
_DN = (((1,), (1,)), ((), ()))


def _matmul(a, w):
    return lax.dot_general(a, w, _DN, precision=_PREC)


_sc_mesh = plsc.VectorSubcoreMesh(core_axis_name="c", subcore_axis_name="s")

_sc_params = pltpu.CompilerParams()
if "needs_layout_passes" in pltpu.CompilerParams.__dataclass_fields__:
    _sc_params = dataclasses.replace(_sc_params, needs_layout_passes=False)


@functools.partial(
    pl.kernel,
    out_type=jax.ShapeDtypeStruct((_NW, _NPAD), jnp.float32),
    mesh=_sc_mesh,
    compiler_params=_sc_params,
    scratch_types=[
        pltpu.VMEM((_NPAD,), jnp.float32),
        pltpu.VMEM((_NB * _CH,), jnp.int32),
        pltpu.VMEM((_CH,), jnp.int32),
    ],
)
def _sc_degree_hist(col_hbm, out_hbm, hist_v, cslab, cext):
    c = lax.axis_index("c")
    s = lax.axis_index("s")
    wid = c * _NSUB + s
    start = _NB * wid + jnp.minimum(wid, _NX)

    @pl.loop(0, _NPAD // 16)
    def _(i):
        hist_v[pl.ds(i * 16, 16)] = jnp.zeros((16,), jnp.float32)

    ones = jnp.ones((16,), jnp.float32)
    pltpu.sync_copy(col_hbm.at[pl.ds(start * _CH, _NB * _CH)], cslab)

    @pl.loop(0, _NB * _CH // 16)
    def _(k):
        idx = cslab[pl.ds(k * 16, 16)]
        plsc.addupdate_scatter(hist_v, [idx], ones)

    @pl.when(wid < _NX)
    def _():
        pltpu.sync_copy(col_hbm.at[pl.ds((start + _NB) * _CH, _CH)], cext)

        @pl.loop(0, _CH // 16)
        def _(k):
            idx = cext[pl.ds(k * 16, 16)]
            plsc.addupdate_scatter(hist_v, [idx], ones)

    pltpu.sync_copy(hist_v, out_hbm.at[wid])


@functools.partial(
    pl.kernel,
    out_type=jax.ShapeDtypeStruct((_NCORES, _NPAD, 128), jnp.float32),
    mesh=_sc_mesh,
    compiler_params=_sc_params,
    scratch_types=[
        [pltpu.VMEM((_CH, 128), jnp.float32)] * 2,
        [pltpu.VMEM((_CH,), jnp.int32)] * 4,
        [pltpu.VMEM((_CH,), jnp.int32)] * 4,
        [pltpu.SemaphoreType.DMA] * 2,
        [pltpu.SemaphoreType.DMA] * 4,
        [pltpu.SemaphoreType.DMA] * 4,
        pltpu.VMEM_SHARED((_NPAD, 128), jnp.float32),
    ],
)
def _sc_edge_pass(y_hbm, row_hbm, col_hbm, trash_r, trash_c, zeros_hbm,
                  out_hbm, gbufs, cbufs, rbufs, gsems, rsems, csems, acc):
    c = lax.axis_index("c")
    s = lax.axis_index("s")
    wid = c * _NSUB + s
    start = _NB * wid + jnp.minimum(wid, _NX)
    cnt = _NB + (wid < _NX).astype(jnp.int32)

    pltpu.sync_copy(zeros_hbm, gbufs[0])
    zrows = _NPAD // _NSUB

    @pl.loop(0, zrows // _CH)
    def _(i):
        pltpu.sync_copy(gbufs[0], acc.at[pl.ds(s * zrows + i * _CH, _CH)])

    plsc.subcore_barrier()

    def copy_idx(k, which, trash, buf, sem):
        e0 = (start + k) * _CH

        @pl.when(k < cnt)
        def _():
            pltpu.async_copy(which.at[pl.ds(e0, _CH)], buf, sem)

        @pl.when(k >= cnt)
        def _():
            pltpu.async_copy(trash, buf, sem)

    def wait_r(b):
        pltpu.make_async_copy(trash_r, rbufs[b], rsems[b]).wait()

    def wait_c(b):
        pltpu.make_async_copy(trash_c, cbufs[b], csems[b]).wait()

    def start_g(b, rb):
        pltpu.async_copy(y_hbm.at[rbufs[rb]], gbufs[b], gsems[b])

    def wait_g(b, rb):
        pltpu.make_async_copy(y_hbm.at[rbufs[rb]], gbufs[b], gsems[b]).wait()

    for k in range(4):
        copy_idx(k, row_hbm, trash_r, rbufs[k], rsems[k])
        copy_idx(k, col_hbm, trash_c, cbufs[k], csems[k])
    for k in range(2):
        wait_r(k)
        start_g(k, k)

    @pl.loop(0, _NCH, step=4)
    def _(j):
        for i in range(4):
            jj = j + i
            b = i % 2
            wait_g(b, i)
            wait_c(i)
            pltpu.sync_copy(gbufs[b], acc.at[cbufs[i]], add=True)

            @pl.when(jj + 4 < _NCH)
            def _():
                copy_idx(jj + 4, row_hbm, trash_r, rbufs[i], rsems[i])
                copy_idx(jj + 4, col_hbm, trash_c, cbufs[i], csems[i])

            @pl.when(jj + 2 < _NCH)
            def _():
                wait_r((i + 2) % 4)
                start_g(b, (i + 2) % 4)

    plsc.subcore_barrier()

    orows = _NPAD // _NSUB
    pltpu.sync_copy(acc.at[pl.ds(s * orows, orows)],
                    out_hbm.at[c, pl.ds(s * orows, orows)])


def _ln(x, scale, bias):
    mu = jnp.mean(x, axis=-1, keepdims=True)
    var = jnp.mean((x - mu) ** 2, axis=-1, keepdims=True)
    return (x - mu) * lax.rsqrt(var + 1e-5) * scale + bias


def _full(shape):
    return pl.BlockSpec(shape, lambda i: (0,) * len(shape))


def _rows(shape):
    return pl.BlockSpec(shape, lambda i: (i,) + (0,) * (len(shape) - 1))


def _embed_body(x_ref, w_ref, b_ref, out_ref):
    out_ref[...] = _matmul(x_ref[...], w_ref[...]) + b_ref[...]


def _dinv_body(hist_ref, out_ref):
    total = jnp.sum(hist_ref[...], axis=0) + 2.0
    out_ref[...] = lax.rsqrt(total)[:, None]


def _pre_body(h_ref, dinv_ref, lns_ref, lnb_ref, pinw_ref, pinb_ref,
              sgus_ref, sgub_ref, gcnw_ref, t_ref, y_ref):
    hn = _ln(h_ref[...], lns_ref[...], lnb_ref[...])
    t = _matmul(hn, pinw_ref[...]) + pinb_ref[...]
    t = 0.5 * t * (1.0 + lax.erf(t * (1.0 / math.sqrt(2.0))))
    g = _ln(t, sgus_ref[...], sgub_ref[...])
    y = dinv_ref[...] * _matmul(g, gcnw_ref[...])
    t_ref[...] = t
    y_ref[...] = y


def _post_body(p_ref, y_ref, t_ref, h_ref, dinv_ref, gcnb_ref,
               poutw_ref, poutb_ref, out_ref):
    acc = p_ref[0] + p_ref[1] + 2.0 * y_ref[...]
    g2 = jnp.tanh(dinv_ref[...] * acc + gcnb_ref[...])
    t2 = g2 * t_ref[...]
    out_ref[...] = h_ref[...] + _matmul(t2, poutw_ref[...]) + poutb_ref[...]


def _final_body(h_ref, w_ref, b_ref, out_ref):
    out_ref[...] = _matmul(h_ref[...], w_ref[...]) + b_ref[...]


_G = _N // _R


def _tc(body, out_shapes, in_specs, out_specs):
    return pl.pallas_call(
        body,
        grid=(_G,),
        in_specs=in_specs,
        out_specs=out_specs,
        out_shape=out_shapes,
    )


def kernel(x, edge_index, batch, emb_W, emb_b, ln_s, ln_b, pin_W, pin_b,
           sgu_s, sgu_b, gcn_W, gcn_b, pout_W, pout_b, out_W, out_b):
    f32 = jnp.float32
    row = edge_index[0]
    col = edge_index[1]

    trash_r = jnp.asarray((np.arange(_CH, dtype=np.int32) * 73) % _N)
    trash_c = jnp.asarray(_N + np.arange(_CH, dtype=np.int32))
    zeros_blk = jnp.zeros((_CH, 128), f32)

    hist = _sc_degree_hist(col)
    dinv = pl.pallas_call(
        _dinv_body,
        out_shape=jax.ShapeDtypeStruct((_NPAD, 1), f32),
    )(hist)

    h = _tc(
        _embed_body,
        jax.ShapeDtypeStruct((_N, 128), f32),
        [_rows((_R, 128)), _full((128, 128)), _full((1, 128))],
        _rows((_R, 128)),
    )(x, emb_W, emb_b[None, :])

    for i in range(2):
        t, y = _tc(
            _pre_body,
            (jax.ShapeDtypeStruct((_N, 128), f32),
             jax.ShapeDtypeStruct((_N, 128), f32)),
            [_rows((_R, 128)), _rows((_R, 1)),
             _full((1, 128)), _full((1, 128)),
             _full((128, 128)), _full((1, 128)),
             _full((1, 128)), _full((1, 128)),
             _full((128, 128))],
            (_rows((_R, 128)), _rows((_R, 128))),
        )(h, dinv, ln_s[i][None, :], ln_b[i][None, :], pin_W[i],
          pin_b[i][None, :], sgu_s[i][None, :], sgu_b[i][None, :],
          gcn_W[i])

        partials = _sc_edge_pass(y, row, col, trash_r, trash_c, zeros_blk)

        h = _tc(
            _post_body,
            jax.ShapeDtypeStruct((_N, 128), f32),
            [pl.BlockSpec((2, _R, 128), lambda i: (0, i, 0)),
             _rows((_R, 128)), _rows((_R, 128)), _rows((_R, 128)),
             _rows((_R, 1)), _full((1, 128)),
             _full((128, 128)), _full((1, 128))],
            _rows((_R, 128)),
        )(partials, y, t, h, dinv, gcn_b[i][None, :], pout_W[i],
          pout_b[i][None, :])

    out = _tc(
        _final_body,
        jax.ShapeDtypeStruct((_N, 64), f32),
        [_rows((_R, 128)), _full((64, 128)), _full((1, 64))],
        _rows((_R, 64)),
    )(h, out_W, out_b[None, :])
    return out


# speedup vs baseline: 1.1226x; 1.0609x over previous
import dataclasses
import functools
import math

import numpy as np

import jax
import jax.numpy as jnp
from jax import lax
from jax.experimental import pallas as pl
from jax.experimental.pallas import tpu as pltpu
from jax.experimental.pallas import tpu_sc as plsc

_N = 10000
_NPAD = 10240
_E = 320000
_NCORES = 2
_NSUB = 16
_NW = _NCORES * _NSUB
_CH = 128
_NRC = _E // _CH
_NB = _NRC // _NW
_NX = _NRC - _NB * _NW
_NCH = _NB + 2
_R = 1000
_PREC = lax.Precision.DEFAULT
_DN = (((1,), (1,)), ((), ()))


def _matmul(a, w):
    return lax.dot_general(a, w, _DN, precision=_PREC)


_sc_mesh = plsc.VectorSubcoreMesh(core_axis_name="c", subcore_axis_name="s")

_sc_params = pltpu.CompilerParams()
if "needs_layout_passes" in pltpu.CompilerParams.__dataclass_fields__:
    _sc_params = dataclasses.replace(_sc_params, needs_layout_passes=False)


@functools.partial(
    pl.kernel,
    out_type=jax.ShapeDtypeStruct((_NW, _NPAD), jnp.float32),
    mesh=_sc_mesh,
    compiler_params=_sc_params,
    scratch_types=[
        pltpu.VMEM((_NPAD,), jnp.float32),
        pltpu.VMEM((_NB * _CH,), jnp.int32),
        pltpu.VMEM((_CH,), jnp.int32),
    ],
)
def _sc_degree_hist(ei_hbm, out_hbm, hist_v, cslab, cext):
    c = lax.axis_index("c")
    s = lax.axis_index("s")
    wid = c * _NSUB + s
    start = _NB * wid + jnp.minimum(wid, _NX)

    @pl.loop(0, _NPAD // 16)
    def _(i):
        hist_v[pl.ds(i * 16, 16)] = jnp.zeros((16,), jnp.float32)

    ones = jnp.ones((16,), jnp.float32)
    pltpu.sync_copy(ei_hbm.at[1, pl.ds(start * _CH, _NB * _CH)], cslab)

    @pl.loop(0, _NB * _CH // 16)
    def _(k):
        idx = cslab[pl.ds(k * 16, 16)]
        plsc.addupdate_scatter(hist_v, [idx], ones)

    @pl.when(wid < _NX)
    def _():
        pltpu.sync_copy(ei_hbm.at[1, pl.ds((start + _NB) * _CH, _CH)], cext)

        @pl.loop(0, _CH // 16)
        def _(k):
            idx = cext[pl.ds(k * 16, 16)]
            plsc.addupdate_scatter(hist_v, [idx], ones)

    pltpu.sync_copy(hist_v, out_hbm.at[wid])


@functools.partial(
    pl.kernel,
    out_type=jax.ShapeDtypeStruct((_NCORES, _NPAD, 128), jnp.float32),
    mesh=_sc_mesh,
    compiler_params=_sc_params,
    scratch_types=[
        [pltpu.VMEM((_CH, 128), jnp.float32)] * 2,
        [pltpu.VMEM((_CH,), jnp.int32)] * 4,
        [pltpu.VMEM((_CH,), jnp.int32)] * 4,
        [pltpu.SemaphoreType.DMA] * 2,
        [pltpu.SemaphoreType.DMA] * 4,
        [pltpu.SemaphoreType.DMA] * 4,
        pltpu.VMEM_SHARED((_NPAD, 128), jnp.float32),
    ],
)
def _sc_edge_pass(y_hbm, ei_hbm, trash_r, trash_c, zeros_hbm,
                  out_hbm, gbufs, cbufs, rbufs, gsems, rsems, csems, acc):
    c = lax.axis_index("c")
    s = lax.axis_index("s")
    wid = c * _NSUB + s
    start = _NB * wid + jnp.minimum(wid, _NX)
    cnt = _NB + (wid < _NX).astype(jnp.int32)

    pltpu.sync_copy(zeros_hbm, gbufs[0])
    zrows = _NPAD // _NSUB

    @pl.loop(0, zrows // _CH)
    def _(i):
        pltpu.sync_copy(gbufs[0], acc.at[pl.ds(s * zrows + i * _CH, _CH)])

    plsc.subcore_barrier()

    def copy_idx(k, which, trash, buf, sem):
        e0 = (start + k) * _CH

        @pl.when(k < cnt)
        def _():
            pltpu.async_copy(ei_hbm.at[which, pl.ds(e0, _CH)], buf, sem)

        @pl.when(k >= cnt)
        def _():
            pltpu.async_copy(trash, buf, sem)

    def wait_r(b):
        pltpu.make_async_copy(trash_r, rbufs[b], rsems[b]).wait()

    def wait_c(b):
        pltpu.make_async_copy(trash_c, cbufs[b], csems[b]).wait()

    def start_g(b, rb):
        pltpu.async_copy(y_hbm.at[rbufs[rb]], gbufs[b], gsems[b])

    def wait_g(b, rb):
        pltpu.make_async_copy(y_hbm.at[rbufs[rb]], gbufs[b], gsems[b]).wait()

    for k in range(4):
        copy_idx(k, 0, trash_r, rbufs[k], rsems[k])
        copy_idx(k, 1, trash_c, cbufs[k], csems[k])
    for k in range(2):
        wait_r(k)
        start_g(k, k)

    @pl.loop(0, _NCH, step=4)
    def _(j):
        for i in range(4):
            jj = j + i
            b = i % 2
            wait_g(b, i)
            wait_c(i)
            pltpu.sync_copy(gbufs[b], acc.at[cbufs[i]], add=True)

            @pl.when(jj + 4 < _NCH)
            def _():
                copy_idx(jj + 4, 0, trash_r, rbufs[i], rsems[i])
                copy_idx(jj + 4, 1, trash_c, cbufs[i], csems[i])

            @pl.when(jj + 2 < _NCH)
            def _():
                wait_r((i + 2) % 4)
                start_g(b, (i + 2) % 4)

    plsc.subcore_barrier()

    orows = _NPAD // _NSUB
    pltpu.sync_copy(acc.at[pl.ds(s * orows, orows)],
                    out_hbm.at[c, pl.ds(s * orows, orows)])


def _ln(x, scale, bias):
    mu = jnp.mean(x, axis=-1, keepdims=True)
    var = jnp.mean((x - mu) ** 2, axis=-1, keepdims=True)
    return (x - mu) * lax.rsqrt(var + 1e-5) * scale + bias


def _full(shape):
    return pl.BlockSpec(shape, lambda i: (0,) * len(shape))


def _rows(shape):
    return pl.BlockSpec(shape, lambda i: (i,) + (0,) * (len(shape) - 1))


def _embed_body(x_ref, w_ref, b_ref, out_ref):
    out_ref[...] = _matmul(x_ref[...], w_ref[...]) + b_ref[...]


def _dinv_body(hist_ref, out_ref):
    total = jnp.sum(hist_ref[...], axis=0) + 2.0
    out_ref[...] = lax.rsqrt(total)[:, None]


def _pre_body(h_ref, dinv_ref, lns_ref, lnb_ref, pinw_ref, pinb_ref,
              sgus_ref, sgub_ref, gcnw_ref, t_ref, y_ref):
    hn = _ln(h_ref[...], lns_ref[...], lnb_ref[...])
    t = _matmul(hn, pinw_ref[...]) + pinb_ref[...]
    t = 0.5 * t * (1.0 + lax.erf(t * (1.0 / math.sqrt(2.0))))
    g = _ln(t, sgus_ref[...], sgub_ref[...])
    y = dinv_ref[...] * _matmul(g, gcnw_ref[...])
    t_ref[...] = t
    y_ref[...] = y


def _post_body(p_ref, y_ref, t_ref, h_ref, dinv_ref, gcnb_ref,
               poutw_ref, poutb_ref, out_ref):
    acc = p_ref[0] + p_ref[1] + 2.0 * y_ref[...]
    g2 = jnp.tanh(dinv_ref[...] * acc + gcnb_ref[...])
    t2 = g2 * t_ref[...]
    out_ref[...] = h_ref[...] + _matmul(t2, poutw_ref[...]) + poutb_ref[...]


def _final_body(h_ref, w_ref, b_ref, out_ref):
    out_ref[...] = _matmul(h_ref[...], w_ref[...]) + b_ref[...]


_G = _N // _R


def _tc(body, out_shapes, in_specs, out_specs):
    return pl.pallas_call(
        body,
        grid=(_G,),
        in_specs=in_specs,
        out_specs=out_specs,
        out_shape=out_shapes,
    )


def kernel(x, edge_index, batch, emb_W, emb_b, ln_s, ln_b, pin_W, pin_b,
           sgu_s, sgu_b, gcn_W, gcn_b, pout_W, pout_b, out_W, out_b):
    f32 = jnp.float32

    trash_r = jnp.asarray((np.arange(_CH, dtype=np.int32) * 73) % _N)
    trash_c = jnp.asarray(_N + np.arange(_CH, dtype=np.int32))
    zeros_blk = jnp.zeros((_CH, 128), f32)

    hist = _sc_degree_hist(edge_index)
    dinv = pl.pallas_call(
        _dinv_body,
        out_shape=jax.ShapeDtypeStruct((_NPAD, 1), f32),
    )(hist)

    h = _tc(
        _embed_body,
        jax.ShapeDtypeStruct((_N, 128), f32),
        [_rows((_R, 128)), _full((128, 128)), _full((1, 128))],
        _rows((_R, 128)),
    )(x, emb_W, emb_b[None, :])

    for i in range(2):
        t, y = _tc(
            _pre_body,
            (jax.ShapeDtypeStruct((_N, 128), f32),
             jax.ShapeDtypeStruct((_N, 128), f32)),
            [_rows((_R, 128)), _rows((_R, 1)),
             _full((1, 128)), _full((1, 128)),
             _full((128, 128)), _full((1, 128)),
             _full((1, 128)), _full((1, 128)),
             _full((128, 128))],
            (_rows((_R, 128)), _rows((_R, 128))),
        )(h, dinv, ln_s[i][None, :], ln_b[i][None, :], pin_W[i],
          pin_b[i][None, :], sgu_s[i][None, :], sgu_b[i][None, :],
          gcn_W[i])

        partials = _sc_edge_pass(y, edge_index, trash_r, trash_c, zeros_blk)

        h = _tc(
            _post_body,
            jax.ShapeDtypeStruct((_N, 128), f32),
            [pl.BlockSpec((2, _R, 128), lambda i: (0, i, 0)),
             _rows((_R, 128)), _rows((_R, 128)), _rows((_R, 128)),
             _rows((_R, 1)), _full((1, 128)),
             _full((128, 128)), _full((1, 128))],
            _rows((_R, 128)),
        )(partials, y, t, h, dinv, gcn_b[i][None, :], pout_W[i],
          pout_b[i][None, :])

    out = _tc(
        _final_body,
        jax.ShapeDtypeStruct((_N, 64), f32),
        [_rows((_R, 128)), _full((64, 128)), _full((1, 64))],
        _rows((_R, 64)),
    )(h, out_W, out_b[None, :])
    return out


# speedup vs baseline: 1.1833x; 1.0541x over previous
import dataclasses
import functools
import math

import numpy as np

import jax
import jax.numpy as jnp
from jax import lax
from jax.experimental import pallas as pl
from jax.experimental.pallas import tpu as pltpu
from jax.experimental.pallas import tpu_sc as plsc

_N = 10000
_NPAD = 10240
_E = 320000
_NCORES = 2
_NSUB = 16
_NW = _NCORES * _NSUB
_CH = 128
_NRC = _E // _CH
_NB = _NRC // _NW
_NX = _NRC - _NB * _NW
_NCH = _NB + 2
_R = 1000
_PREC = lax.Precision.DEFAULT
_DN = (((1,), (1,)), ((), ()))


def _matmul(a, w):
    return lax.dot_general(a, w, _DN, precision=_PREC)


_sc_mesh = plsc.VectorSubcoreMesh(core_axis_name="c", subcore_axis_name="s")

_sc_params = pltpu.CompilerParams()
if "needs_layout_passes" in pltpu.CompilerParams.__dataclass_fields__:
    _sc_params = dataclasses.replace(_sc_params, needs_layout_passes=False)


@functools.partial(
    pl.kernel,
    out_type=jax.ShapeDtypeStruct((_NW, _NPAD), jnp.float32),
    mesh=_sc_mesh,
    compiler_params=_sc_params,
    scratch_types=[
        pltpu.VMEM((_NPAD,), jnp.float32),
        pltpu.VMEM((_NB * _CH,), jnp.int32),
        pltpu.VMEM((_CH,), jnp.int32),
    ],
)
def _sc_degree_hist(ei_hbm, out_hbm, hist_v, cslab, cext):
    c = lax.axis_index("c")
    s = lax.axis_index("s")
    wid = c * _NSUB + s
    start = _NB * wid + jnp.minimum(wid, _NX)

    @pl.loop(0, _NPAD // 16)
    def _(i):
        hist_v[pl.ds(i * 16, 16)] = jnp.zeros((16,), jnp.float32)

    ones = jnp.ones((16,), jnp.float32)
    pltpu.sync_copy(ei_hbm.at[1, pl.ds(start * _CH, _NB * _CH)], cslab)

    @pl.loop(0, _NB * _CH // 16)
    def _(k):
        idx = cslab[pl.ds(k * 16, 16)]
        plsc.addupdate_scatter(hist_v, [idx], ones)

    @pl.when(wid < _NX)
    def _():
        pltpu.sync_copy(ei_hbm.at[1, pl.ds((start + _NB) * _CH, _CH)], cext)

        @pl.loop(0, _CH // 16)
        def _(k):
            idx = cext[pl.ds(k * 16, 16)]
            plsc.addupdate_scatter(hist_v, [idx], ones)

    pltpu.sync_copy(hist_v, out_hbm.at[wid])


@functools.partial(
    pl.kernel,
    out_type=jax.ShapeDtypeStruct((_NCORES, _NPAD, 128), jnp.float32),
    mesh=_sc_mesh,
    compiler_params=_sc_params,
    scratch_types=[
        [pltpu.VMEM((_CH, 128), jnp.float32)] * 2,
        [pltpu.VMEM((_CH,), jnp.int32)] * 4,
        [pltpu.VMEM((_CH,), jnp.int32)] * 4,
        [pltpu.SemaphoreType.DMA] * 2,
        [pltpu.SemaphoreType.DMA] * 4,
        [pltpu.SemaphoreType.DMA] * 4,
        pltpu.VMEM_SHARED((_NPAD, 128), jnp.float32),
    ],
)
def _sc_edge_pass(y_hbm, ei_hbm, trash_r, trash_c, zeros_hbm,
                  out_hbm, gbufs, cbufs, rbufs, gsems, rsems, csems, acc):
    c = lax.axis_index("c")
    s = lax.axis_index("s")
    wid = c * _NSUB + s
    start = _NB * wid + jnp.minimum(wid, _NX)
    cnt = _NB + (wid < _NX).astype(jnp.int32)

    pltpu.sync_copy(zeros_hbm, gbufs[0])
    zrows = _NPAD // _NSUB

    @pl.loop(0, zrows // _CH)
    def _(i):
        pltpu.sync_copy(gbufs[0], acc.at[pl.ds(s * zrows + i * _CH, _CH)])

    plsc.subcore_barrier()

    def copy_idx(k, which, trash, buf, sem):
        e0 = (start + k) * _CH

        @pl.when(k < cnt)
        def _():
            pltpu.async_copy(ei_hbm.at[which, pl.ds(e0, _CH)], buf, sem)

        @pl.when(k >= cnt)
        def _():
            pltpu.async_copy(trash, buf, sem)

    def wait_r(b):
        pltpu.make_async_copy(trash_r, rbufs[b], rsems[b]).wait()

    def wait_c(b):
        pltpu.make_async_copy(trash_c, cbufs[b], csems[b]).wait()

    def start_g(b, rb):
        pltpu.async_copy(y_hbm.at[rbufs[rb]], gbufs[b], gsems[b])

    def wait_g(b, rb):
        pltpu.make_async_copy(y_hbm.at[rbufs[rb]], gbufs[b], gsems[b]).wait()

    for k in range(4):
        copy_idx(k, 0, trash_r, rbufs[k], rsems[k])
        copy_idx(k, 1, trash_c, cbufs[k], csems[k])
    for k in range(2):
        wait_r(k)
        start_g(k, k)

    @pl.loop(0, _NCH, step=4)
    def _(j):
        for i in range(4):
            jj = j + i
            b = i % 2
            wait_g(b, i)
            wait_c(i)
            pltpu.sync_copy(gbufs[b], acc.at[cbufs[i]], add=True)

            @pl.when(jj + 4 < _NCH)
            def _():
                copy_idx(jj + 4, 0, trash_r, rbufs[i], rsems[i])
                copy_idx(jj + 4, 1, trash_c, cbufs[i], csems[i])

            @pl.when(jj + 2 < _NCH)
            def _():
                wait_r((i + 2) % 4)
                start_g(b, (i + 2) % 4)

    plsc.subcore_barrier()

    orows = _NPAD // _NSUB
    pltpu.sync_copy(acc.at[pl.ds(s * orows, orows)],
                    out_hbm.at[c, pl.ds(s * orows, orows)])


def _ln(x, scale, bias):
    mu = jnp.mean(x, axis=-1, keepdims=True)
    var = jnp.mean((x - mu) ** 2, axis=-1, keepdims=True)
    return (x - mu) * lax.rsqrt(var + 1e-5) * scale + bias


def _full(shape):
    return pl.BlockSpec(shape, lambda i: (0,) * len(shape))


def _rows(shape):
    return pl.BlockSpec(shape, lambda i: (i,) + (0,) * (len(shape) - 1))


def _embed_body(x_ref, w_ref, b_ref, out_ref):
    out_ref[...] = _matmul(x_ref[...], w_ref[...]) + b_ref[...]


def _dinv_body(hist_ref, out_ref):
    total = jnp.sum(hist_ref[...], axis=0) + 2.0
    out_ref[...] = lax.rsqrt(total)[:, None]


def _pre_body(h_ref, dinv_ref, lns_ref, lnb_ref, pinw_ref, pinb_ref,
              sgus_ref, sgub_ref, gcnw_ref, t_ref, y_ref):
    t, y = _pre(h_ref[...], dinv_ref, lns_ref, lnb_ref, pinw_ref, pinb_ref,
                sgus_ref, sgub_ref, gcnw_ref)
    t_ref[...] = t
    y_ref[...] = y


def _post(p_ref, y_ref, t_ref, h_ref, dinv_ref, gcnb_ref, poutw_ref,
          poutb_ref):
    acc = p_ref[0] + p_ref[1] + 2.0 * y_ref[...]
    g2 = jnp.tanh(dinv_ref[...] * acc + gcnb_ref[...])
    t2 = g2 * t_ref[...]
    return h_ref[...] + _matmul(t2, poutw_ref[...]) + poutb_ref[...]


def _pre(h, dinv_ref, lns_ref, lnb_ref, pinw_ref, pinb_ref, sgus_ref,
         sgub_ref, gcnw_ref):
    hn = _ln(h, lns_ref[...], lnb_ref[...])
    t = _matmul(hn, pinw_ref[...]) + pinb_ref[...]
    t = 0.5 * t * (1.0 + lax.erf(t * (1.0 / math.sqrt(2.0))))
    g = _ln(t, sgus_ref[...], sgub_ref[...])
    y = dinv_ref[...] * _matmul(g, gcnw_ref[...])
    return t, y


def _post_pre_body(p_ref, y_ref, t_ref, h_ref, dinv_ref, gcnb_ref,
                   poutw_ref, poutb_ref, lns_ref, lnb_ref, pinw_ref,
                   pinb_ref, sgus_ref, sgub_ref, gcnw_ref,
                   h_out, t_out, y_out):
    h2 = _post(p_ref, y_ref, t_ref, h_ref, dinv_ref, gcnb_ref,
               poutw_ref, poutb_ref)
    h_out[...] = h2
    t2, y2 = _pre(h2, dinv_ref, lns_ref, lnb_ref, pinw_ref, pinb_ref,
                  sgus_ref, sgub_ref, gcnw_ref)
    t_out[...] = t2
    y_out[...] = y2


def _post_final_body(p_ref, y_ref, t_ref, h_ref, dinv_ref, gcnb_ref,
                     poutw_ref, poutb_ref, outw_ref, outb_ref, out_ref):
    h2 = _post(p_ref, y_ref, t_ref, h_ref, dinv_ref, gcnb_ref,
               poutw_ref, poutb_ref)
    out_ref[...] = _matmul(h2, outw_ref[...]) + outb_ref[...]


_G = _N // _R


def _tc(body, out_shapes, in_specs, out_specs):
    return pl.pallas_call(
        body,
        grid=(_G,),
        in_specs=in_specs,
        out_specs=out_specs,
        out_shape=out_shapes,
    )


def kernel(x, edge_index, batch, emb_W, emb_b, ln_s, ln_b, pin_W, pin_b,
           sgu_s, sgu_b, gcn_W, gcn_b, pout_W, pout_b, out_W, out_b):
    f32 = jnp.float32

    trash_r = jnp.asarray((np.arange(_CH, dtype=np.int32) * 73) % _N)
    trash_c = jnp.asarray(_N + np.arange(_CH, dtype=np.int32))
    zeros_blk = jnp.zeros((_CH, 128), f32)

    hist = _sc_degree_hist(edge_index)
    dinv = pl.pallas_call(
        _dinv_body,
        out_shape=jax.ShapeDtypeStruct((_NPAD, 1), f32),
    )(hist)

    h = _tc(
        _embed_body,
        jax.ShapeDtypeStruct((_N, 128), f32),
        [_rows((_R, 128)), _full((128, 128)), _full((1, 128))],
        _rows((_R, 128)),
    )(x, emb_W, emb_b[None, :])

    t, y = _tc(
        _pre_body,
        (jax.ShapeDtypeStruct((_N, 128), f32),
         jax.ShapeDtypeStruct((_N, 128), f32)),
        [_rows((_R, 128)), _rows((_R, 1)),
         _full((1, 128)), _full((1, 128)),
         _full((128, 128)), _full((1, 128)),
         _full((1, 128)), _full((1, 128)),
         _full((128, 128))],
        (_rows((_R, 128)), _rows((_R, 128))),
    )(h, dinv, ln_s[0][None, :], ln_b[0][None, :], pin_W[0],
      pin_b[0][None, :], sgu_s[0][None, :], sgu_b[0][None, :], gcn_W[0])

    partials = _sc_edge_pass(y, edge_index, trash_r, trash_c, zeros_blk)

    h, t, y = _tc(
        _post_pre_body,
        (jax.ShapeDtypeStruct((_N, 128), f32),
         jax.ShapeDtypeStruct((_N, 128), f32),
         jax.ShapeDtypeStruct((_N, 128), f32)),
        [pl.BlockSpec((2, _R, 128), lambda i: (0, i, 0)),
         _rows((_R, 128)), _rows((_R, 128)), _rows((_R, 128)),
         _rows((_R, 1)), _full((1, 128)),
         _full((128, 128)), _full((1, 128)),
         _full((1, 128)), _full((1, 128)),
         _full((128, 128)), _full((1, 128)),
         _full((1, 128)), _full((1, 128)),
         _full((128, 128))],
        (_rows((_R, 128)), _rows((_R, 128)), _rows((_R, 128))),
    )(partials, y, t, h, dinv, gcn_b[0][None, :], pout_W[0],
      pout_b[0][None, :], ln_s[1][None, :], ln_b[1][None, :], pin_W[1],
      pin_b[1][None, :], sgu_s[1][None, :], sgu_b[1][None, :], gcn_W[1])

    partials = _sc_edge_pass(y, edge_index, trash_r, trash_c, zeros_blk)

    out = _tc(
        _post_final_body,
        jax.ShapeDtypeStruct((_N, 64), f32),
        [pl.BlockSpec((2, _R, 128), lambda i: (0, i, 0)),
         _rows((_R, 128)), _rows((_R, 128)), _rows((_R, 128)),
         _rows((_R, 1)), _full((1, 128)),
         _full((128, 128)), _full((1, 128)),
         _full((64, 128)), _full((1, 64))],
        _rows((_R, 64)),
    )(partials, y, t, h, dinv, gcn_b[1][None, :], pout_W[1],
      pout_b[1][None, :], out_W, out_b[None, :])
    return out
